# baseline (speedup 1.0000x reference)
import functools

import jax
import jax.numpy as jnp
from jax.experimental import pallas as pl
from jax.experimental.pallas import tpu as pltpu


def _bottleneck_body(x_ref, w1_ref, s1_ref, b1_ref,
                     w2_ref, s2_ref, b2_ref,
                     w3_ref, s3_ref, b3_ref,
                     o_ref, *, Nb, H, W, Cin, P):
    rows = Nb * H * W
    cdt = jnp.bfloat16

    x = x_ref[...].reshape(rows, Cin)
    h1 = jnp.dot(x.astype(cdt), w1_ref[...],
                 preferred_element_type=jnp.float32)
    h1 = jnp.maximum(h1 * s1_ref[...] + b1_ref[...], 0.0)

    h1m = h1.astype(cdt).reshape(Nb, H, W, P)
    zcol = jnp.zeros((Nb, H, 1, P), cdt)
    left = jnp.concatenate([zcol, h1m[:, :, :W - 1, :]], axis=2)
    right = jnp.concatenate([h1m[:, :, 1:, :], zcol], axis=2)
    hcat = jnp.concatenate([left, h1m, right], axis=3)

    zrow = jnp.zeros((Nb, 1, W, 3 * P), cdt)
    win0 = jnp.concatenate([zrow, hcat[:, :H - 1]], axis=1)
    win2 = jnp.concatenate([hcat[:, 1:], zrow], axis=1)

    acc = jnp.dot(win0.reshape(rows, 3 * P), w2_ref[0],
                  preferred_element_type=jnp.float32)
    acc = acc + jnp.dot(hcat.reshape(rows, 3 * P), w2_ref[1],
                        preferred_element_type=jnp.float32)
    acc = acc + jnp.dot(win2.reshape(rows, 3 * P), w2_ref[2],
                        preferred_element_type=jnp.float32)
    h2 = jnp.maximum(acc * s2_ref[...] + b2_ref[...], 0.0)

    h3 = jnp.dot(h2.astype(cdt), w3_ref[...],
                 preferred_element_type=jnp.float32)
    h3 = h3 * s3_ref[...] + b3_ref[...]

    res = x_ref[...].reshape(rows, Cin).astype(jnp.float32)
    out = jnp.maximum(h3 + res, 0.0)
    o_ref[...] = out.reshape(Nb, H, W, Cin).astype(o_ref.dtype)


def kernel(x_nhwc, w1, s1, b1, w2, s2, b2, w3, s3, b3):
    N, H, W, Cin = x_nhwc.shape
    P = w1.shape[1]

    Nb = 1

    w1c = w1.astype(jnp.bfloat16)
    w2c = w2.reshape(3, 3 * P, P).astype(jnp.bfloat16)
    w3c = w3.astype(jnp.bfloat16)

    full = lambda a: pl.BlockSpec(a.shape, lambda n: (0,) * a.ndim)
    body = functools.partial(_bottleneck_body, Nb=Nb, H=H, W=W, Cin=Cin, P=P)

    return pl.pallas_call(
        body,
        out_shape=jax.ShapeDtypeStruct((N, H, W, Cin), x_nhwc.dtype),
        grid=(N // Nb,),
        in_specs=[
            pl.BlockSpec((Nb, H, W, Cin), lambda n: (n, 0, 0, 0)),
            full(w1c), full(s1), full(b1),
            full(w2c), full(s2), full(b2),
            full(w3c), full(s3), full(b3),
        ],
        out_specs=pl.BlockSpec((Nb, H, W, Cin), lambda n: (n, 0, 0, 0)),
        compiler_params=pltpu.CompilerParams(
            dimension_semantics=("parallel",),
            vmem_limit_bytes=48 * 1024 * 1024),
    )(x_nhwc,
      w1c, s1, b1,
      w2c, s2, b2,
      w3c, s3, b3)


# speedup vs baseline: 3.4706x; 3.4706x over previous
import functools

import jax
import jax.numpy as jnp
from jax.experimental import pallas as pl
from jax.experimental.pallas import tpu as pltpu


def _body(x_ref, xt_ref, xb_ref, w1_ref, s1_ref, b1_ref,
          w2_ref, s2_ref, b2_ref, w3_ref, s3_ref, b3_ref,
          o_ref, *, HB, G, W, N, Cin, P):
    g = pl.program_id(0) * (G // 2) + pl.program_id(1)
    rows = HB * W * N
    hrow = W * N

    xm = x_ref[...].reshape(rows, Cin)
    xt = xt_ref[...].reshape(hrow, Cin)
    xb = xb_ref[...].reshape(hrow, Cin)
    w1 = w1_ref[...]

    def conv1(v):
        h = jnp.dot(v, w1, preferred_element_type=jnp.float32)
        return jnp.maximum(h * s1_ref[...] + b1_ref[...], 0.0)

    h1_mid = conv1(xm)
    h1_top = jnp.where(g == 0, 0.0, conv1(xt))
    h1_bot = jnp.where(g == G - 1, 0.0, conv1(xb))

    h1 = jnp.concatenate([h1_top, h1_mid, h1_bot], axis=0)
    h1m = h1.reshape(HB + 2, W, N, P)
    zcol = jnp.zeros((HB + 2, 1, N, P), h1m.dtype)
    left = jnp.concatenate([zcol, h1m[:, :W - 1]], axis=1)
    right = jnp.concatenate([h1m[:, 1:], zcol], axis=1)
    hcat = jnp.concatenate([left, h1m, right], axis=3)

    acc = jnp.dot(hcat[:HB].reshape(rows, 3 * P), w2_ref[0],
                  preferred_element_type=jnp.float32)
    acc = acc + jnp.dot(hcat[1:HB + 1].reshape(rows, 3 * P), w2_ref[1],
                        preferred_element_type=jnp.float32)
    acc = acc + jnp.dot(hcat[2:].reshape(rows, 3 * P), w2_ref[2],
                        preferred_element_type=jnp.float32)
    h2 = jnp.maximum(acc * s2_ref[...] + b2_ref[...], 0.0)

    h3 = jnp.dot(h2, w3_ref[...], preferred_element_type=jnp.float32)
    h3 = h3 * s3_ref[...] + b3_ref[...]
    out = jnp.maximum(h3 + xm, 0.0)
    o_ref[...] = out.reshape(HB, hrow, Cin).astype(o_ref.dtype)


def kernel(x_nhwc, w1, s1, b1, w2, s2, b2, w3, s3, b3):
    N, H, W, Cin = x_nhwc.shape
    P = w1.shape[1]
    HB = 7
    G = H // HB

    x3 = jnp.transpose(x_nhwc, (1, 2, 0, 3)).reshape(H, W * N, Cin)
    w2c = w2.reshape(3, 3 * P, P)

    full = lambda a: pl.BlockSpec(a.shape, lambda c, j: (0,) * a.ndim)
    body = functools.partial(_body, HB=HB, G=G, W=W, N=N, Cin=Cin, P=P)

    G2 = G // 2
    band = lambda c, j: c * G2 + j
    out3 = pl.pallas_call(
        body,
        out_shape=jax.ShapeDtypeStruct((H, W * N, Cin), x_nhwc.dtype),
        grid=(2, G2),
        in_specs=[
            pl.BlockSpec((HB, W * N, Cin), lambda c, j: (band(c, j), 0, 0)),
            pl.BlockSpec((1, W * N, Cin),
                         lambda c, j: (jnp.maximum(band(c, j) * HB - 1, 0), 0, 0)),
            pl.BlockSpec((1, W * N, Cin),
                         lambda c, j: (jnp.minimum(band(c, j) * HB + HB, H - 1), 0, 0)),
            full(w1), full(s1), full(b1),
            full(w2c), full(s2), full(b2),
            full(w3), full(s3), full(b3),
        ],
        out_specs=pl.BlockSpec((HB, W * N, Cin), lambda c, j: (band(c, j), 0, 0)),
        compiler_params=pltpu.CompilerParams(
            dimension_semantics=("arbitrary", "arbitrary"),
            vmem_limit_bytes=55 * 1024 * 1024),
    )(x3, x3, x3,
      w1, s1, b1,
      w2c, s2, b2,
      w3, s3, b3)

    return jnp.transpose(out3.reshape(H, W, N, Cin), (2, 0, 1, 3))


# speedup vs baseline: 3.4834x; 1.0037x over previous
import functools

import jax
import jax.numpy as jnp
from jax.experimental import pallas as pl
from jax.experimental.pallas import tpu as pltpu


def _body(x_ref, xt_ref, xb_ref, w1_ref, s1_ref, b1_ref,
          w2_ref, s2_ref, b2_ref, w3_ref, s3_ref, b3_ref,
          o_ref, *, HB, G, W, N, Cin, P):
    g = pl.program_id(0) * (G // 2) + pl.program_id(1)
    rows = HB * W * N
    hrow = W * N

    xm = x_ref[...].reshape(rows, Cin)
    xt = xt_ref[...].reshape(hrow, Cin)
    xb = xb_ref[...].reshape(hrow, Cin)
    w1 = w1_ref[...]

    def conv1(v):
        h = jnp.dot(v, w1, preferred_element_type=jnp.float32)
        return jnp.maximum(h * s1_ref[...] + b1_ref[...], 0.0)

    h1_mid = conv1(xm)
    h1_top = jnp.where(g == 0, 0.0, conv1(xt))
    h1_bot = jnp.where(g == G - 1, 0.0, conv1(xb))

    h1 = jnp.concatenate([h1_top, h1_mid, h1_bot], axis=0).astype(jnp.bfloat16)
    h1m = h1.reshape(HB + 2, W, N, P)
    zcol = jnp.zeros((HB + 2, 1, N, P), h1m.dtype)
    left = jnp.concatenate([zcol, h1m[:, :W - 1]], axis=1)
    right = jnp.concatenate([h1m[:, 1:], zcol], axis=1)
    hcat = jnp.concatenate([left, h1m, right], axis=3)

    w2 = w2_ref[...].astype(jnp.bfloat16)
    acc = jnp.dot(hcat[:HB].reshape(rows, 3 * P), w2[0],
                  preferred_element_type=jnp.float32)
    acc = acc + jnp.dot(hcat[1:HB + 1].reshape(rows, 3 * P), w2[1],
                        preferred_element_type=jnp.float32)
    acc = acc + jnp.dot(hcat[2:].reshape(rows, 3 * P), w2[2],
                        preferred_element_type=jnp.float32)
    h2 = jnp.maximum(acc * s2_ref[...] + b2_ref[...], 0.0)

    h3 = jnp.dot(h2.astype(jnp.bfloat16), w3_ref[...].astype(jnp.bfloat16),
                 preferred_element_type=jnp.float32)
    h3 = h3 * s3_ref[...] + b3_ref[...]
    out = jnp.maximum(h3 + xm, 0.0)
    o_ref[...] = out.reshape(HB, hrow, Cin).astype(o_ref.dtype)


def kernel(x_nhwc, w1, s1, b1, w2, s2, b2, w3, s3, b3):
    N, H, W, Cin = x_nhwc.shape
    P = w1.shape[1]
    HB = 7
    G = H // HB

    x3 = jnp.transpose(x_nhwc, (1, 2, 0, 3)).reshape(H, W * N, Cin)
    w2c = w2.reshape(3, 3 * P, P)

    full = lambda a: pl.BlockSpec(a.shape, lambda c, j: (0,) * a.ndim)
    body = functools.partial(_body, HB=HB, G=G, W=W, N=N, Cin=Cin, P=P)

    G2 = G // 2
    band = lambda c, j: c * G2 + j
    out3 = pl.pallas_call(
        body,
        out_shape=jax.ShapeDtypeStruct((H, W * N, Cin), x_nhwc.dtype),
        grid=(2, G2),
        in_specs=[
            pl.BlockSpec((HB, W * N, Cin), lambda c, j: (band(c, j), 0, 0)),
            pl.BlockSpec((1, W * N, Cin),
                         lambda c, j: (jnp.maximum(band(c, j) * HB - 1, 0), 0, 0)),
            pl.BlockSpec((1, W * N, Cin),
                         lambda c, j: (jnp.minimum(band(c, j) * HB + HB, H - 1), 0, 0)),
            full(w1), full(s1), full(b1),
            full(w2c), full(s2), full(b2),
            full(w3), full(s3), full(b3),
        ],
        out_specs=pl.BlockSpec((HB, W * N, Cin), lambda c, j: (band(c, j), 0, 0)),
        compiler_params=pltpu.CompilerParams(
            dimension_semantics=("arbitrary", "arbitrary"),
            vmem_limit_bytes=55 * 1024 * 1024),
    )(x3, x3, x3,
      w1, s1, b1,
      w2c, s2, b2,
      w3, s3, b3)

    return jnp.transpose(out3.reshape(H, W, N, Cin), (2, 0, 1, 3))
